# baseline (speedup 1.0000x reference)
import functools

import jax
import jax.numpy as jnp
from jax import lax
from jax.experimental import pallas as pl
from jax.experimental.pallas import tpu as pltpu
from jax.experimental.pallas import tpu_sc as plsc

N = 10000
E = 320000
D = 128
NC = 2
NS = 16
NW = NC * NS
C = 64
K = (E + NW * C - 1) // (NW * C)
EPAD = NW * C * K
NPAD = 10112
RPS = NPAD // NS
NPADG = 10240
RPSG = NPADG // NS
BR = 2000


def _sc_mesh():
    return plsc.VectorSubcoreMesh(core_axis_name="c", subcore_axis_name="s",
                                  num_cores=NC, num_subcores=NS)


@functools.partial(
    pl.kernel,
    mesh=_sc_mesh(),
    out_type=jax.ShapeDtypeStruct((NC, NPADG), jnp.float32),
    scratch_types=[
        pltpu.VMEM((K, 2, C), jnp.int32),
        pltpu.VMEM((C,), jnp.float32),
        pltpu.VMEM_SHARED((NPADG,), jnp.float32),
    ],
)
def _sc_degree(eidx_hbm, ones_hbm, zeros1_hbm, out_hbm, idx_v, ones_v, deg_sh):
    cid = lax.axis_index("c")
    sid = lax.axis_index("s")
    wid = sid * NC + cid
    pltpu.sync_copy(zeros1_hbm.at[pl.ds(sid * RPSG, RPSG)],
                    deg_sh.at[pl.ds(sid * RPSG, RPSG)])
    pltpu.sync_copy(ones_hbm, ones_v)
    pltpu.sync_copy(eidx_hbm.at[wid], idx_v)
    plsc.subcore_barrier()

    def body(j, _):
        pltpu.sync_copy(ones_v, deg_sh.at[idx_v.at[j, 1]], add=True)
        return 0

    lax.fori_loop(0, K, body, 0)
    plsc.subcore_barrier()
    pltpu.sync_copy(deg_sh.at[pl.ds(sid * RPSG, RPSG)],
                    out_hbm.at[cid, pl.ds(sid * RPSG, RPSG)])


NBUF = 5
GA = NBUF - 1
NIDX = 10
IDX_AHEAD = 7


@functools.partial(
    pl.kernel,
    mesh=_sc_mesh(),
    out_type=jax.ShapeDtypeStruct((NC, NPAD, D), jnp.float32),
    scratch_types=[
        pltpu.VMEM((2 * NIDX, C), jnp.int32),
        pltpu.VMEM((NBUF, C, D), jnp.float32),
        pltpu.VMEM_SHARED((NPAD, D), jnp.float32),
        pltpu.SemaphoreType.DMA((NIDX,)),
        pltpu.SemaphoreType.DMA((NBUF,)),
        pltpu.SemaphoreType.DMA((NBUF,)),
    ],
)
def _sc_edge_pass(y_hbm, eidx_hbm, zeros_hbm, out_hbm,
                  idx_v, rows_v, acc_sh, isem, gsem, ssem):
    cid = lax.axis_index("c")
    sid = lax.axis_index("s")
    wid = sid * NC + cid
    pltpu.sync_copy(zeros_hbm.at[pl.ds(sid * RPS, RPS)],
                    acc_sh.at[pl.ds(sid * RPS, RPS)])

    def fire_idx(j):
        s = pl.multiple_of((j % NIDX) * 2, 2)
        pltpu.async_copy(eidx_hbm.at[wid, j], idx_v.at[pl.ds(s, 2)],
                         isem.at[j % NIDX])

    def fire_gather(j, b):
        s = pl.multiple_of((j % NIDX) * 2, 2)
        pltpu.make_async_copy(eidx_hbm.at[wid, j],
                              idx_v.at[pl.ds(s, 2)],
                              isem.at[j % NIDX]).wait()
        pltpu.async_copy(y_hbm.at[idx_v.at[(j % NIDX) * 2]],
                         rows_v.at[b], gsem.at[b])

    for j0 in range(IDX_AHEAD):
        fire_idx(j0)
    for j0 in range(GA):
        fire_gather(j0, j0)
    plsc.subcore_barrier()

    def body(jj, _):
        for b in range(NBUF):
            j = jj * NBUF + b

            @pl.when(j < K)
            def _():
                pltpu.make_async_copy(y_hbm.at[idx_v.at[(j % NIDX) * 2]],
                                      rows_v.at[b], gsem.at[b]).wait()
                dst_row = idx_v.at[(j % NIDX) * 2 + 1]

                @pl.when(j < K - NBUF)
                def _():
                    pltpu.async_copy(rows_v.at[b], acc_sh.at[dst_row],
                                     ssem.at[b], add=True)

                @pl.when(j >= K - NBUF)
                def _():
                    pltpu.sync_copy(rows_v.at[b], acc_sh.at[dst_row],
                                    add=True)

                ji = j + IDX_AHEAD

                @pl.when(ji < K)
                def _():
                    fire_idx(ji)

                jr = j + GA
                br = (b + GA) % NBUF

                @pl.when(jr < K)
                def _():
                    @pl.when(jr >= NBUF)
                    def _():
                        pltpu.make_async_copy(
                            rows_v.at[br],
                            acc_sh.at[idx_v.at[((jr - NBUF) % NIDX) * 2 + 1]],
                            ssem.at[br]).wait()

                    fire_gather(jr, br)
        return 0

    lax.fori_loop(0, (K + NBUF - 1) // NBUF, body, 0)
    plsc.subcore_barrier()
    pltpu.sync_copy(acc_sh.at[pl.ds(sid * RPS, RPS)],
                    out_hbm.at[cid, pl.ds(sid * RPS, RPS)])


def _dis_block(degp_ref):
    return lax.rsqrt(degp_ref[0, 0, :] + degp_ref[0, 1, :] + 1.0)[:, None]


def _mm_scale_body(x_ref, w_ref, degp_ref, y_ref):
    xw = jnp.dot(x_ref[...], w_ref[...], preferred_element_type=jnp.float32)
    y_ref[...] = xw * _dis_block(degp_ref)


def _layer2_body(p_ref, y1_ref, degp_ref, b1_ref, w34_ref, y2_ref):
    dis = _dis_block(degp_ref)
    tot = p_ref[0] + p_ref[1] + y1_ref[...]
    h = jnp.maximum(dis * tot + b1_ref[...][None, :], 0.0)
    y2_ref[...] = jnp.dot(h, w34_ref[...],
                          preferred_element_type=jnp.float32) * dis


def _final_body(q_ref, y2_ref, degp_ref, b34_ref, mean_ref, std_ref):
    dis = _dis_block(degp_ref)
    m = dis * (q_ref[0] + q_ref[1] + y2_ref[...]) + b34_ref[...][None, :]
    mean_ref[...] = m[:, :64]
    std_ref[...] = m[:, 64:]


_GRID = N // BR
_row_spec = pl.BlockSpec((BR, D), lambda i: (i, 0))
_full2_spec = pl.BlockSpec((1, 2, BR), lambda i: (i, 0, 0))
_w_spec = pl.BlockSpec((D, D), lambda i: (0, 0))
_b_spec = pl.BlockSpec((D,), lambda i: (0,))
_part_spec = pl.BlockSpec((2, BR, D), lambda i: (0, i, 0))
_half_spec = pl.BlockSpec((BR, 64), lambda i: (i, 0))

_mm_scale = pl.pallas_call(
    _mm_scale_body,
    grid=(_GRID,),
    in_specs=[_row_spec, _w_spec, _full2_spec],
    out_specs=_row_spec,
    out_shape=jax.ShapeDtypeStruct((N, D), jnp.float32),
)

_layer2 = pl.pallas_call(
    _layer2_body,
    grid=(_GRID,),
    in_specs=[_part_spec, _row_spec, _full2_spec, _b_spec, _w_spec],
    out_specs=_row_spec,
    out_shape=jax.ShapeDtypeStruct((N, D), jnp.float32),
)

_final = pl.pallas_call(
    _final_body,
    grid=(_GRID,),
    in_specs=[_part_spec, _row_spec, _full2_spec, _b_spec],
    out_specs=[_half_spec, _half_spec],
    out_shape=[jax.ShapeDtypeStruct((N, 64), jnp.float32),
               jax.ShapeDtypeStruct((N, 64), jnp.float32)],
)


def kernel(x, edge_index, W1, b1, W3, b3, W4, b4):
    pad = EPAD - E
    ar = jnp.arange(pad, dtype=jnp.int32)
    src = jnp.concatenate([edge_index[0], ar % N]).reshape(NW, K, C)
    dst = jnp.concatenate([edge_index[1], N + (ar % NS)]).reshape(NW, K, C)
    eidx = jnp.stack([src, dst], axis=2)
    ones_c = jnp.ones((C,), jnp.float32)
    zeros1 = jnp.zeros((NPADG,), jnp.float32)
    zeros2 = jnp.zeros((NPAD, D), jnp.float32)
    W34 = jnp.concatenate([W3, W4], axis=1)
    b34 = jnp.concatenate([b3, b4])

    degp = _sc_degree(eidx, ones_c, zeros1)
    degp_r = degp[:, :N].reshape(2, _GRID, BR).transpose(1, 0, 2)
    y1 = _mm_scale(x, W1, degp_r)
    p = _sc_edge_pass(y1, eidx, zeros2)
    y2 = _layer2(p, y1, degp_r, b1, W34)
    q = _sc_edge_pass(y2, eidx, zeros2)
    mean, std = _final(q, y2, degp_r, b34)
    return (mean, std)


# speedup vs baseline: 1.0016x; 1.0016x over previous
import functools

import jax
import jax.numpy as jnp
from jax import lax
from jax.experimental import pallas as pl
from jax.experimental.pallas import tpu as pltpu
from jax.experimental.pallas import tpu_sc as plsc

N = 10000
E = 320000
D = 128
NC = 2
NS = 16
NW = NC * NS
C = 112
K = (E + NW * C - 1) // (NW * C)
EPAD = NW * C * K
NPAD = 10112
RPS = NPAD // NS
NPADG = 10240
RPSG = NPADG // NS
BR = 2000


def _sc_mesh():
    return plsc.VectorSubcoreMesh(core_axis_name="c", subcore_axis_name="s",
                                  num_cores=NC, num_subcores=NS)


@functools.partial(
    pl.kernel,
    mesh=_sc_mesh(),
    out_type=jax.ShapeDtypeStruct((NC, NPADG), jnp.float32),
    scratch_types=[
        pltpu.VMEM((K, 2, C), jnp.int32),
        pltpu.VMEM((C,), jnp.float32),
        pltpu.VMEM_SHARED((NPADG,), jnp.float32),
    ],
)
def _sc_degree(eidx_hbm, ones_hbm, zeros1_hbm, out_hbm, idx_v, ones_v, deg_sh):
    cid = lax.axis_index("c")
    sid = lax.axis_index("s")
    wid = sid * NC + cid
    pltpu.sync_copy(zeros1_hbm.at[pl.ds(sid * RPSG, RPSG)],
                    deg_sh.at[pl.ds(sid * RPSG, RPSG)])
    pltpu.sync_copy(ones_hbm, ones_v)
    pltpu.sync_copy(eidx_hbm.at[wid], idx_v)
    plsc.subcore_barrier()

    def body(j, _):
        pltpu.sync_copy(ones_v, deg_sh.at[idx_v.at[j, 1]], add=True)
        return 0

    lax.fori_loop(0, K, body, 0)
    plsc.subcore_barrier()
    pltpu.sync_copy(deg_sh.at[pl.ds(sid * RPSG, RPSG)],
                    out_hbm.at[cid, pl.ds(sid * RPSG, RPSG)])


NBUF = 3
GA = NBUF - 1
NIDX = 8
IDX_AHEAD = 5


@functools.partial(
    pl.kernel,
    mesh=_sc_mesh(),
    out_type=jax.ShapeDtypeStruct((NC, NPAD, D), jnp.float32),
    scratch_types=[
        pltpu.VMEM((2 * NIDX, C), jnp.int32),
        pltpu.VMEM((NBUF, C, D), jnp.float32),
        pltpu.VMEM_SHARED((NPAD, D), jnp.float32),
        pltpu.SemaphoreType.DMA((NIDX,)),
        pltpu.SemaphoreType.DMA((NBUF,)),
        pltpu.SemaphoreType.DMA((NBUF,)),
    ],
)
def _sc_edge_pass(y_hbm, eidx_hbm, zeros_hbm, out_hbm,
                  idx_v, rows_v, acc_sh, isem, gsem, ssem):
    cid = lax.axis_index("c")
    sid = lax.axis_index("s")
    wid = sid * NC + cid
    pltpu.sync_copy(zeros_hbm.at[pl.ds(sid * RPS, RPS)],
                    acc_sh.at[pl.ds(sid * RPS, RPS)])

    def fire_idx(j):
        s = pl.multiple_of((j % NIDX) * 2, 2)
        pltpu.async_copy(eidx_hbm.at[wid, j], idx_v.at[pl.ds(s, 2)],
                         isem.at[j % NIDX])

    def fire_gather(j, b):
        s = pl.multiple_of((j % NIDX) * 2, 2)
        pltpu.make_async_copy(eidx_hbm.at[wid, j],
                              idx_v.at[pl.ds(s, 2)],
                              isem.at[j % NIDX]).wait()
        pltpu.async_copy(y_hbm.at[idx_v.at[(j % NIDX) * 2]],
                         rows_v.at[b], gsem.at[b])

    for j0 in range(IDX_AHEAD):
        fire_idx(j0)
    for j0 in range(GA):
        fire_gather(j0, j0)
    plsc.subcore_barrier()

    def body(jj, _):
        for b in range(NBUF):
            j = jj * NBUF + b

            @pl.when(j < K)
            def _():
                pltpu.make_async_copy(y_hbm.at[idx_v.at[(j % NIDX) * 2]],
                                      rows_v.at[b], gsem.at[b]).wait()
                dst_row = idx_v.at[(j % NIDX) * 2 + 1]

                @pl.when(j < K - NBUF)
                def _():
                    pltpu.async_copy(rows_v.at[b], acc_sh.at[dst_row],
                                     ssem.at[b], add=True)

                @pl.when(j >= K - NBUF)
                def _():
                    pltpu.sync_copy(rows_v.at[b], acc_sh.at[dst_row],
                                    add=True)

                ji = j + IDX_AHEAD

                @pl.when(ji < K)
                def _():
                    fire_idx(ji)

                jr = j + GA
                br = (b + GA) % NBUF

                @pl.when(jr < K)
                def _():
                    @pl.when(jr >= NBUF)
                    def _():
                        pltpu.make_async_copy(
                            rows_v.at[br],
                            acc_sh.at[idx_v.at[((jr - NBUF) % NIDX) * 2 + 1]],
                            ssem.at[br]).wait()

                    fire_gather(jr, br)
        return 0

    lax.fori_loop(0, (K + NBUF - 1) // NBUF, body, 0)
    plsc.subcore_barrier()
    pltpu.sync_copy(acc_sh.at[pl.ds(sid * RPS, RPS)],
                    out_hbm.at[cid, pl.ds(sid * RPS, RPS)])


def _dis_block(degp_ref):
    return lax.rsqrt(degp_ref[0, 0, :] + degp_ref[0, 1, :] + 1.0)[:, None]


def _mm_scale_body(x_ref, w_ref, degp_ref, y_ref):
    xw = jnp.dot(x_ref[...], w_ref[...], preferred_element_type=jnp.float32)
    y_ref[...] = xw * _dis_block(degp_ref)


def _layer2_body(p_ref, y1_ref, degp_ref, b1_ref, w34_ref, y2_ref):
    dis = _dis_block(degp_ref)
    tot = p_ref[0] + p_ref[1] + y1_ref[...]
    h = jnp.maximum(dis * tot + b1_ref[...][None, :], 0.0)
    y2_ref[...] = jnp.dot(h, w34_ref[...],
                          preferred_element_type=jnp.float32) * dis


def _final_body(q_ref, y2_ref, degp_ref, b34_ref, mean_ref, std_ref):
    dis = _dis_block(degp_ref)
    m = dis * (q_ref[0] + q_ref[1] + y2_ref[...]) + b34_ref[...][None, :]
    mean_ref[...] = m[:, :64]
    std_ref[...] = m[:, 64:]


_GRID = N // BR
_row_spec = pl.BlockSpec((BR, D), lambda i: (i, 0))
_full2_spec = pl.BlockSpec((1, 2, BR), lambda i: (i, 0, 0))
_w_spec = pl.BlockSpec((D, D), lambda i: (0, 0))
_b_spec = pl.BlockSpec((D,), lambda i: (0,))
_part_spec = pl.BlockSpec((2, BR, D), lambda i: (0, i, 0))
_half_spec = pl.BlockSpec((BR, 64), lambda i: (i, 0))

_mm_scale = pl.pallas_call(
    _mm_scale_body,
    grid=(_GRID,),
    in_specs=[_row_spec, _w_spec, _full2_spec],
    out_specs=_row_spec,
    out_shape=jax.ShapeDtypeStruct((N, D), jnp.float32),
)

_layer2 = pl.pallas_call(
    _layer2_body,
    grid=(_GRID,),
    in_specs=[_part_spec, _row_spec, _full2_spec, _b_spec, _w_spec],
    out_specs=_row_spec,
    out_shape=jax.ShapeDtypeStruct((N, D), jnp.float32),
)

_final = pl.pallas_call(
    _final_body,
    grid=(_GRID,),
    in_specs=[_part_spec, _row_spec, _full2_spec, _b_spec],
    out_specs=[_half_spec, _half_spec],
    out_shape=[jax.ShapeDtypeStruct((N, 64), jnp.float32),
               jax.ShapeDtypeStruct((N, 64), jnp.float32)],
)


def kernel(x, edge_index, W1, b1, W3, b3, W4, b4):
    pad = EPAD - E
    ar = jnp.arange(pad, dtype=jnp.int32)
    src = jnp.concatenate([edge_index[0], ar % N]).reshape(NW, K, C)
    dst = jnp.concatenate([edge_index[1], N + (ar % NS)]).reshape(NW, K, C)
    eidx = jnp.stack([src, dst], axis=2)
    ones_c = jnp.ones((C,), jnp.float32)
    zeros1 = jnp.zeros((NPADG,), jnp.float32)
    zeros2 = jnp.zeros((NPAD, D), jnp.float32)
    W34 = jnp.concatenate([W3, W4], axis=1)
    b34 = jnp.concatenate([b3, b4])

    degp = _sc_degree(eidx, ones_c, zeros1)
    degp_r = degp[:, :N].reshape(2, _GRID, BR).transpose(1, 0, 2)
    y1 = _mm_scale(x, W1, degp_r)
    p = _sc_edge_pass(y1, eidx, zeros2)
    y2 = _layer2(p, y1, degp_r, b1, W34)
    q = _sc_edge_pass(y2, eidx, zeros2)
    mean, std = _final(q, y2, degp_r, b34)
    return (mean, std)


# speedup vs baseline: 1.0141x; 1.0125x over previous
import functools

import jax
import jax.numpy as jnp
from jax import lax
from jax.experimental import pallas as pl
from jax.experimental.pallas import tpu as pltpu
from jax.experimental.pallas import tpu_sc as plsc

N = 10000
E = 320000
D = 128
NC = 2
NS = 16
NW = NC * NS
C = 96
K = (E + NW * C - 1) // (NW * C)
EPAD = NW * C * K
NPAD = 10112
RPS = NPAD // NS
NPADG = 10240
RPSG = NPADG // NS
BR = 2000


def _sc_mesh():
    return plsc.VectorSubcoreMesh(core_axis_name="c", subcore_axis_name="s",
                                  num_cores=NC, num_subcores=NS)


@functools.partial(
    pl.kernel,
    mesh=_sc_mesh(),
    out_type=jax.ShapeDtypeStruct((NC, NPADG), jnp.float32),
    scratch_types=[
        pltpu.VMEM((K, 2, C), jnp.int32),
        pltpu.VMEM((C,), jnp.float32),
        pltpu.VMEM_SHARED((NPADG,), jnp.float32),
    ],
)
def _sc_degree(eidx_hbm, ones_hbm, zeros1_hbm, out_hbm, idx_v, ones_v, deg_sh):
    cid = lax.axis_index("c")
    sid = lax.axis_index("s")
    wid = sid * NC + cid
    pltpu.sync_copy(zeros1_hbm.at[pl.ds(sid * RPSG, RPSG)],
                    deg_sh.at[pl.ds(sid * RPSG, RPSG)])
    pltpu.sync_copy(ones_hbm, ones_v)
    pltpu.sync_copy(eidx_hbm.at[wid], idx_v)
    plsc.subcore_barrier()

    def body(j, _):
        pltpu.sync_copy(ones_v, deg_sh.at[idx_v.at[j, 1]], add=True)
        return 0

    lax.fori_loop(0, K, body, 0)
    plsc.subcore_barrier()
    pltpu.sync_copy(deg_sh.at[pl.ds(sid * RPSG, RPSG)],
                    out_hbm.at[cid, pl.ds(sid * RPSG, RPSG)])


NBUF = 3
GA = NBUF - 1
NIDX = 8
IDX_AHEAD = 5


@functools.partial(
    pl.kernel,
    mesh=_sc_mesh(),
    out_type=jax.ShapeDtypeStruct((NC, NPAD, D), jnp.float32),
    scratch_types=[
        pltpu.VMEM((2 * NIDX, C), jnp.int32),
        pltpu.VMEM((NBUF, C, D), jnp.float32),
        pltpu.VMEM_SHARED((NPAD, D), jnp.float32),
        pltpu.SemaphoreType.DMA((NIDX,)),
        pltpu.SemaphoreType.DMA((NBUF,)),
        pltpu.SemaphoreType.DMA((NBUF,)),
    ],
)
def _sc_edge_pass(y_hbm, eidx_hbm, zeros_hbm, out_hbm,
                  idx_v, rows_v, acc_sh, isem, gsem, ssem):
    cid = lax.axis_index("c")
    sid = lax.axis_index("s")
    wid = sid * NC + cid
    pltpu.sync_copy(zeros_hbm.at[pl.ds(sid * RPS, RPS)],
                    acc_sh.at[pl.ds(sid * RPS, RPS)])

    def fire_idx(j):
        s = pl.multiple_of((j % NIDX) * 2, 2)
        pltpu.async_copy(eidx_hbm.at[wid, j], idx_v.at[pl.ds(s, 2)],
                         isem.at[j % NIDX])

    def fire_gather(j, b):
        s = pl.multiple_of((j % NIDX) * 2, 2)
        pltpu.make_async_copy(eidx_hbm.at[wid, j],
                              idx_v.at[pl.ds(s, 2)],
                              isem.at[j % NIDX]).wait()
        pltpu.async_copy(y_hbm.at[idx_v.at[(j % NIDX) * 2]],
                         rows_v.at[b], gsem.at[b])

    for j0 in range(IDX_AHEAD):
        fire_idx(j0)
    for j0 in range(GA):
        fire_gather(j0, j0)
    plsc.subcore_barrier()

    def body(jj, _):
        for b in range(NBUF):
            j = jj * NBUF + b

            @pl.when(j < K)
            def _():
                pltpu.make_async_copy(y_hbm.at[idx_v.at[(j % NIDX) * 2]],
                                      rows_v.at[b], gsem.at[b]).wait()
                dst_row = idx_v.at[(j % NIDX) * 2 + 1]

                @pl.when(j < K - NBUF)
                def _():
                    pltpu.async_copy(rows_v.at[b], acc_sh.at[dst_row],
                                     ssem.at[b], add=True)

                @pl.when(j >= K - NBUF)
                def _():
                    pltpu.sync_copy(rows_v.at[b], acc_sh.at[dst_row],
                                    add=True)

                ji = j + IDX_AHEAD

                @pl.when(ji < K)
                def _():
                    fire_idx(ji)

                jr = j + GA
                br = (b + GA) % NBUF

                @pl.when(jr < K)
                def _():
                    @pl.when(jr >= NBUF)
                    def _():
                        pltpu.make_async_copy(
                            rows_v.at[br],
                            acc_sh.at[idx_v.at[((jr - NBUF) % NIDX) * 2 + 1]],
                            ssem.at[br]).wait()

                    fire_gather(jr, br)
        return 0

    lax.fori_loop(0, (K + NBUF - 1) // NBUF, body, 0)
    plsc.subcore_barrier()
    pltpu.sync_copy(acc_sh.at[pl.ds(sid * RPS, RPS)],
                    out_hbm.at[cid, pl.ds(sid * RPS, RPS)])


def _dis_block(degp_ref):
    return lax.rsqrt(degp_ref[0, 0, :] + degp_ref[0, 1, :] + 1.0)[:, None]


def _mm_scale_body(x_ref, w_ref, degp_ref, y_ref):
    xw = jnp.dot(x_ref[...], w_ref[...], preferred_element_type=jnp.float32)
    y_ref[...] = xw * _dis_block(degp_ref)


def _layer2_body(p_ref, y1_ref, degp_ref, b1_ref, w34_ref, y2_ref):
    dis = _dis_block(degp_ref)
    tot = p_ref[0] + p_ref[1] + y1_ref[...]
    h = jnp.maximum(dis * tot + b1_ref[...][None, :], 0.0)
    y2_ref[...] = jnp.dot(h, w34_ref[...],
                          preferred_element_type=jnp.float32) * dis


def _final_body(q_ref, y2_ref, degp_ref, b34_ref, mean_ref, std_ref):
    dis = _dis_block(degp_ref)
    m = dis * (q_ref[0] + q_ref[1] + y2_ref[...]) + b34_ref[...][None, :]
    mean_ref[...] = m[:, :64]
    std_ref[...] = m[:, 64:]


_GRID = N // BR
_row_spec = pl.BlockSpec((BR, D), lambda i: (i, 0))
_full2_spec = pl.BlockSpec((1, 2, BR), lambda i: (i, 0, 0))
_w_spec = pl.BlockSpec((D, D), lambda i: (0, 0))
_b_spec = pl.BlockSpec((D,), lambda i: (0,))
_part_spec = pl.BlockSpec((2, BR, D), lambda i: (0, i, 0))
_half_spec = pl.BlockSpec((BR, 64), lambda i: (i, 0))

_mm_scale = pl.pallas_call(
    _mm_scale_body,
    grid=(_GRID,),
    in_specs=[_row_spec, _w_spec, _full2_spec],
    out_specs=_row_spec,
    out_shape=jax.ShapeDtypeStruct((N, D), jnp.float32),
)

_layer2 = pl.pallas_call(
    _layer2_body,
    grid=(_GRID,),
    in_specs=[_part_spec, _row_spec, _full2_spec, _b_spec, _w_spec],
    out_specs=_row_spec,
    out_shape=jax.ShapeDtypeStruct((N, D), jnp.float32),
)

_final = pl.pallas_call(
    _final_body,
    grid=(_GRID,),
    in_specs=[_part_spec, _row_spec, _full2_spec, _b_spec],
    out_specs=[_half_spec, _half_spec],
    out_shape=[jax.ShapeDtypeStruct((N, 64), jnp.float32),
               jax.ShapeDtypeStruct((N, 64), jnp.float32)],
)


def kernel(x, edge_index, W1, b1, W3, b3, W4, b4):
    pad = EPAD - E
    ar = jnp.arange(pad, dtype=jnp.int32)
    src = jnp.concatenate([edge_index[0], ar % N]).reshape(NW, K, C)
    dst = jnp.concatenate([edge_index[1], N + (ar % NS)]).reshape(NW, K, C)
    eidx = jnp.stack([src, dst], axis=2)
    ones_c = jnp.ones((C,), jnp.float32)
    zeros1 = jnp.zeros((NPADG,), jnp.float32)
    zeros2 = jnp.zeros((NPAD, D), jnp.float32)
    W34 = jnp.concatenate([W3, W4], axis=1)
    b34 = jnp.concatenate([b3, b4])

    degp = _sc_degree(eidx, ones_c, zeros1)
    degp_r = degp[:, :N].reshape(2, _GRID, BR).transpose(1, 0, 2)
    y1 = _mm_scale(x, W1, degp_r)
    p = _sc_edge_pass(y1, eidx, zeros2)
    y2 = _layer2(p, y1, degp_r, b1, W34)
    q = _sc_edge_pass(y2, eidx, zeros2)
    mean, std = _final(q, y2, degp_r, b34)
    return (mean, std)
